# baseline (speedup 1.0000x reference)
import functools

import jax
import jax.numpy as jnp
from jax import lax
from jax.experimental import pallas as pl
from jax.experimental.pallas import tpu as pltpu
from jax.experimental.pallas import tpu_sc as plsc

_NC, _NS = 2, 16
_NW = _NC * _NS
_ROWS = 4 * 4096
_D = 2048
_CH = 8


def _sc_copy(x_hbm, o_hbm, buf0, buf1, sem0, sem1):
    wid = lax.axis_index("s") * _NC + lax.axis_index("c")
    rows_per_w = _ROWS // _NW
    base = wid * rows_per_w
    n_chunks = rows_per_w // _CH
    bufs = (buf0, buf1)
    sems = (sem0, sem1)

    def in_copy(ci, slot):
        return pltpu.make_async_copy(
            x_hbm.at[pl.ds(base + ci * _CH, _CH)], bufs[slot], sems[slot]
        )

    def out_copy(ci, slot):
        return pltpu.make_async_copy(
            bufs[slot], o_hbm.at[pl.ds(base + ci * _CH, _CH)], sems[slot]
        )

    in_copy(0, 0).start()

    def body(i, _):
        ci0 = 2 * i
        in_copy(ci0 + 1, 1).start()
        in_copy(ci0, 0).wait()
        out_copy(ci0, 0).start()
        out_copy(ci0, 0).wait()

        @pl.when(ci0 + 2 < n_chunks)
        def _():
            in_copy(ci0 + 2, 0).start()

        in_copy(ci0 + 1, 1).wait()
        out_copy(ci0 + 1, 1).start()
        out_copy(ci0 + 1, 1).wait()
        return 0

    lax.fori_loop(0, n_chunks // 2, body, 0)


def kernel(inputs, medians):
    del medians
    B, S, D = inputs.shape
    x = inputs.reshape(B * S, D)
    run = pl.kernel(
        _sc_copy,
        out_type=jax.ShapeDtypeStruct((B * S, D), jnp.float32),
        mesh=plsc.VectorSubcoreMesh(core_axis_name="c", subcore_axis_name="s"),
        scratch_types=[
            pltpu.VMEM((_CH, _D), jnp.float32),
            pltpu.VMEM((_CH, _D), jnp.float32),
            pltpu.SemaphoreType.DMA,
            pltpu.SemaphoreType.DMA,
        ],
    )
    return run(x).reshape(B, S, D)
